# baseline (speedup 1.0000x reference)
import jax
import jax.numpy as jnp
from jax.experimental import pallas as pl
from jax.experimental.pallas import tpu as pltpu

NG = 128
FX = 128
H = 128
FU = 128
FOUT = 2


def _pick_blk(n):
    for blk in (10000, 4000, 2000, 1000, 500, 250, 200, 125, 100, 50, 25, 20,
                10, 8, 5, 4, 2, 1):
        if n % blk == 0:
            return blk
    return n


_HI = jax.lax.Precision.HIGHEST
_MED = jax.lax.Precision.DEFAULT


def _seg_scatter(oh, h):
    h_lo = h - h.astype(jnp.bfloat16).astype(jnp.float32)
    h_ll = h_lo - h_lo.astype(jnp.bfloat16).astype(jnp.float32)
    return (jnp.dot(oh, h, preferred_element_type=jnp.float32)
            + jnp.dot(oh, h_lo, preferred_element_type=jnp.float32)
            + jnp.dot(oh, h_ll, preferred_element_type=jnp.float32))


def _mlp3(x, w0, b0, w1, b1, w2, b2, prec=_HI):
    y = jnp.maximum(
        jnp.dot(x, w0, preferred_element_type=jnp.float32, precision=prec) + b0,
        0.0)
    y = jnp.maximum(
        jnp.dot(y, w1, preferred_element_type=jnp.float32, precision=prec) + b1,
        0.0)
    return jnp.dot(y, w2, preferred_element_type=jnp.float32, precision=prec) + b2


def _body(nb, blk,
          x1_ref, x2_ref, b1_ref, b2_ref,
          e10, e1b0, e11, e1b1, e12, e1b2,
          r10, r1b0, r11, r1b1, r12, r1b2,
          e2a, e2b, e2b0, e21, e2b1, e22, e2b2,
          r20, r2b0, r21, r2b1, r22, r2b2,
          m0, mb0, m1, mb1, m2, mb2,
          o_ref, agg1, u1s, agg2):
    p = pl.program_id(0)
    i = pl.program_id(1)

    @pl.when(jnp.logical_and(p == 0, i == 0))
    def _():
        agg1[...] = jnp.zeros_like(agg1)
        agg2[...] = jnp.zeros_like(agg2)

    @pl.when(p == 0)
    def _():
        h = _mlp3(x1_ref[...], e10[...], e1b0[...], e11[...], e1b1[...],
                  e12[...], e1b2[...], prec=_MED)
        ids = b1_ref[...].reshape(1, blk)
        oh = (jax.lax.broadcasted_iota(jnp.int32, (NG, blk), 0) == ids
              ).astype(jnp.float32)
        agg1[...] += _seg_scatter(oh, h)

    @pl.when(jnp.logical_and(p == 1, i == 0))
    def _():
        u1 = _mlp3(agg1[...], r10[...], r1b0[...], r11[...], r1b1[...],
                   r12[...], r1b2[...], prec=_MED)
        u1s[...] = u1.astype(jnp.bfloat16).astype(jnp.float32)

    @pl.when(p == 1)
    def _():
        ids1 = b1_ref[...].reshape(1, blk)
        oh1 = (jax.lax.broadcasted_iota(jnp.int32, (NG, blk), 0) == ids1
               ).astype(jnp.float32)
        u1g = jax.lax.dot_general(oh1, u1s[...], (((0,), (0,)), ((), ())),
                                  preferred_element_type=jnp.float32)
        y = jnp.dot(x2_ref[...], e2a[...], preferred_element_type=jnp.float32,
                    precision=_MED)
        y = y + jnp.dot(u1g, e2b[...], preferred_element_type=jnp.float32,
                        precision=_MED)
        y = jnp.maximum(y + e2b0[...], 0.0)
        y = jnp.maximum(jnp.dot(y, e21[...], preferred_element_type=jnp.float32,
                                precision=_MED) + e2b1[...], 0.0)
        h2 = jnp.dot(y, e22[...], preferred_element_type=jnp.float32,
                     precision=_MED) + e2b2[...]
        ids2 = b2_ref[...].reshape(1, blk)
        oh2 = (jax.lax.broadcasted_iota(jnp.int32, (NG, blk), 0) == ids2
               ).astype(jnp.float32)
        agg2[...] += _seg_scatter(oh2, h2)

    @pl.when(jnp.logical_and(p == 1, i == nb - 1))
    def _():
        u2 = _mlp3(agg2[...], r20[...], r2b0[...], r21[...], r2b1[...],
                   r22[...], r2b2[...], prec=_MED)
        o_ref[...] = _mlp3(u2, m0[...], mb0[...], m1[...], mb1[...],
                           m2[...], mb2[...], prec=_MED)


def kernel(x1, x2, batch1, batch2,
           ds1_emb_W0, ds1_emb_b0, ds1_emb_W1, ds1_emb_b1, ds1_emb_W2, ds1_emb_b2,
           ds1_red_W0, ds1_red_b0, ds1_red_W1, ds1_red_b1, ds1_red_W2, ds1_red_b2,
           ds2_emb_W0, ds2_emb_b0, ds2_emb_W1, ds2_emb_b1, ds2_emb_W2, ds2_emb_b2,
           ds2_red_W0, ds2_red_b0, ds2_red_W1, ds2_red_b1, ds2_red_W2, ds2_red_b2,
           mlp_W0, mlp_b0, mlp_W1, mlp_b1, mlp_W2, mlp_b2):
    n = x1.shape[0]
    blk = _pick_blk(n)
    nb = n // blk

    b1r = batch1.reshape(nb, 1, blk)
    b2r = batch2.reshape(nb, 1, blk)

    def row(b):
        return b.reshape(1, -1)

    w0a = ds2_emb_W0[:FX]
    w0b = ds2_emb_W0[FX:]

    m2p = jnp.zeros((FU, 128), jnp.float32).at[:, :FOUT].set(mlp_W2)
    mb2p = jnp.zeros((1, 128), jnp.float32).at[:, :FOUT].set(mlp_b2)

    def full(a):
        return pl.BlockSpec(a.shape, lambda p, i: (0,) * a.ndim)

    weights = [ds1_emb_W0, row(ds1_emb_b0), ds1_emb_W1, row(ds1_emb_b1),
               ds1_emb_W2, row(ds1_emb_b2),
               ds1_red_W0, row(ds1_red_b0), ds1_red_W1, row(ds1_red_b1),
               ds1_red_W2, row(ds1_red_b2),
               w0a, w0b, row(ds2_emb_b0), ds2_emb_W1, row(ds2_emb_b1),
               ds2_emb_W2, row(ds2_emb_b2),
               ds2_red_W0, row(ds2_red_b0), ds2_red_W1, row(ds2_red_b1),
               ds2_red_W2, row(ds2_red_b2),
               mlp_W0, row(mlp_b0), mlp_W1, row(mlp_b1), m2p, mb2p]

    in_specs = [
        pl.BlockSpec((blk, FX), lambda p, i: (jnp.where(p == 0, i, 0), 0)),
        pl.BlockSpec((blk, FX), lambda p, i: (jnp.where(p == 1, i, 0), 0)),
        pl.BlockSpec((1, 1, blk), lambda p, i: (i, 0, 0)),
        pl.BlockSpec((1, 1, blk), lambda p, i: (i, 0, 0)),
    ] + [full(w) for w in weights]

    import functools
    out = pl.pallas_call(
        functools.partial(_body, nb, blk),
        grid=(2, nb),
        in_specs=in_specs,
        out_specs=pl.BlockSpec((NG, 128), lambda p, i: (0, 0)),
        out_shape=jax.ShapeDtypeStruct((NG, 128), jnp.float32),
        scratch_shapes=[
            pltpu.VMEM((NG, H), jnp.float32),
            pltpu.VMEM((NG, FU), jnp.float32),
            pltpu.VMEM((NG, H), jnp.float32),
        ],
        compiler_params=pltpu.CompilerParams(
            dimension_semantics=("arbitrary", "arbitrary"),
        ),
    )(x1, x2, b1r, b2r, *weights)
    return out[:, :FOUT]


# speedup vs baseline: 1.0728x; 1.0728x over previous
import jax
import jax.numpy as jnp
from jax.experimental import pallas as pl
from jax.experimental.pallas import tpu as pltpu

NG = 128
FX = 128
H = 128
FU = 128
FOUT = 2


def _pick_blk(n):
    for blk in (10000, 4000, 2000, 1000, 500, 250, 200, 125, 100, 50, 25, 20,
                10, 8, 5, 4, 2, 1):
        if n % blk == 0:
            return blk
    return n


_HI = jax.lax.Precision.HIGHEST
_MED = jax.lax.Precision.DEFAULT


def _seg_scatter(oh, h, passes=3):
    h_lo = h - h.astype(jnp.bfloat16).astype(jnp.float32)
    out = (jnp.dot(oh, h, preferred_element_type=jnp.float32)
           + jnp.dot(oh, h_lo, preferred_element_type=jnp.float32))
    if passes >= 3:
        h_ll = h_lo - h_lo.astype(jnp.bfloat16).astype(jnp.float32)
        out = out + jnp.dot(oh, h_ll, preferred_element_type=jnp.float32)
    return out


def _mlp3(x, w0, b0, w1, b1, w2, b2, prec=_HI):
    y = jnp.maximum(
        jnp.dot(x, w0, preferred_element_type=jnp.float32, precision=prec) + b0,
        0.0)
    y = jnp.maximum(
        jnp.dot(y, w1, preferred_element_type=jnp.float32, precision=prec) + b1,
        0.0)
    return jnp.dot(y, w2, preferred_element_type=jnp.float32, precision=prec) + b2


def _body(nb, blk,
          x1_ref, x2_ref, b1_ref, b2_ref,
          e10, e1b0, e11, e1b1, e12, e1b2,
          r10, r1b0, r11, r1b1, r12, r1b2,
          e2a, e2b, e2b0, e21, e2b1, e22, e2b2,
          r20, r2b0, r21, r2b1, r22, r2b2,
          m0, mb0, m1, mb1, m2, mb2,
          o_ref, agg1, u1s, agg2):
    p = pl.program_id(0)
    i = pl.program_id(1)

    @pl.when(jnp.logical_and(p == 0, i == 0))
    def _():
        agg1[...] = jnp.zeros_like(agg1)
        agg2[...] = jnp.zeros_like(agg2)

    @pl.when(p == 0)
    def _():
        h = _mlp3(x1_ref[...], e10[...], e1b0[...], e11[...], e1b1[...],
                  e12[...], e1b2[...], prec=_MED)
        ids = b1_ref[...].reshape(1, blk)
        oh = (jax.lax.broadcasted_iota(jnp.int32, (NG, blk), 0) == ids
              ).astype(jnp.float32)
        agg1[...] += _seg_scatter(oh, h)

    @pl.when(jnp.logical_and(p == 1, i == 0))
    def _():
        u1 = _mlp3(agg1[...], r10[...], r1b0[...], r11[...], r1b1[...],
                   r12[...], r1b2[...], prec=_MED)
        u1s[...] = u1.astype(jnp.bfloat16).astype(jnp.float32)

    @pl.when(p == 1)
    def _():
        ids1 = b1_ref[...].reshape(1, blk)
        oh1 = (jax.lax.broadcasted_iota(jnp.int32, (NG, blk), 0) == ids1
               ).astype(jnp.float32)
        u1g = jax.lax.dot_general(oh1, u1s[...], (((0,), (0,)), ((), ())),
                                  preferred_element_type=jnp.float32)
        y = jnp.dot(x2_ref[...], e2a[...], preferred_element_type=jnp.float32,
                    precision=_MED)
        y = y + jnp.dot(u1g, e2b[...], preferred_element_type=jnp.float32,
                        precision=_MED)
        y = jnp.maximum(y + e2b0[...], 0.0)
        y = jnp.maximum(jnp.dot(y, e21[...], preferred_element_type=jnp.float32,
                                precision=_MED) + e2b1[...], 0.0)
        h2 = jnp.dot(y, e22[...], preferred_element_type=jnp.float32,
                     precision=_MED) + e2b2[...]
        ids2 = b2_ref[...].reshape(1, blk)
        oh2 = (jax.lax.broadcasted_iota(jnp.int32, (NG, blk), 0) == ids2
               ).astype(jnp.float32)
        agg2[...] += _seg_scatter(oh2, h2, passes=2)

    @pl.when(jnp.logical_and(p == 1, i == nb - 1))
    def _():
        u2 = _mlp3(agg2[...], r20[...], r2b0[...], r21[...], r2b1[...],
                   r22[...], r2b2[...], prec=_MED)
        o_ref[...] = _mlp3(u2, m0[...], mb0[...], m1[...], mb1[...],
                           m2[...], mb2[...], prec=_MED)


def kernel(x1, x2, batch1, batch2,
           ds1_emb_W0, ds1_emb_b0, ds1_emb_W1, ds1_emb_b1, ds1_emb_W2, ds1_emb_b2,
           ds1_red_W0, ds1_red_b0, ds1_red_W1, ds1_red_b1, ds1_red_W2, ds1_red_b2,
           ds2_emb_W0, ds2_emb_b0, ds2_emb_W1, ds2_emb_b1, ds2_emb_W2, ds2_emb_b2,
           ds2_red_W0, ds2_red_b0, ds2_red_W1, ds2_red_b1, ds2_red_W2, ds2_red_b2,
           mlp_W0, mlp_b0, mlp_W1, mlp_b1, mlp_W2, mlp_b2):
    n = x1.shape[0]
    blk = _pick_blk(n)
    nb = n // blk

    b1r = batch1.reshape(nb, 1, blk)
    b2r = batch2.reshape(nb, 1, blk)

    def row(b):
        return b.reshape(1, -1)

    w0a = ds2_emb_W0[:FX]
    w0b = ds2_emb_W0[FX:]

    m2p = jnp.zeros((FU, 128), jnp.float32).at[:, :FOUT].set(mlp_W2)
    mb2p = jnp.zeros((1, 128), jnp.float32).at[:, :FOUT].set(mlp_b2)

    def full(a):
        return pl.BlockSpec(a.shape, lambda p, i: (0,) * a.ndim)

    weights = [ds1_emb_W0, row(ds1_emb_b0), ds1_emb_W1, row(ds1_emb_b1),
               ds1_emb_W2, row(ds1_emb_b2),
               ds1_red_W0, row(ds1_red_b0), ds1_red_W1, row(ds1_red_b1),
               ds1_red_W2, row(ds1_red_b2),
               w0a, w0b, row(ds2_emb_b0), ds2_emb_W1, row(ds2_emb_b1),
               ds2_emb_W2, row(ds2_emb_b2),
               ds2_red_W0, row(ds2_red_b0), ds2_red_W1, row(ds2_red_b1),
               ds2_red_W2, row(ds2_red_b2),
               mlp_W0, row(mlp_b0), mlp_W1, row(mlp_b1), m2p, mb2p]

    in_specs = [
        pl.BlockSpec((blk, FX), lambda p, i: (jnp.where(p == 0, i, 0), 0)),
        pl.BlockSpec((blk, FX), lambda p, i: (jnp.where(p == 1, i, 0), 0)),
        pl.BlockSpec((1, 1, blk), lambda p, i: (i, 0, 0)),
        pl.BlockSpec((1, 1, blk), lambda p, i: (i, 0, 0)),
    ] + [full(w) for w in weights]

    import functools
    out = pl.pallas_call(
        functools.partial(_body, nb, blk),
        grid=(2, nb),
        in_specs=in_specs,
        out_specs=pl.BlockSpec((NG, 128), lambda p, i: (0, 0)),
        out_shape=jax.ShapeDtypeStruct((NG, 128), jnp.float32),
        scratch_shapes=[
            pltpu.VMEM((NG, H), jnp.float32),
            pltpu.VMEM((NG, FU), jnp.float32),
            pltpu.VMEM((NG, H), jnp.float32),
        ],
        compiler_params=pltpu.CompilerParams(
            dimension_semantics=("arbitrary", "arbitrary"),
        ),
    )(x1, x2, b1r, b2r, *weights)
    return out[:, :FOUT]


# speedup vs baseline: 1.4898x; 1.3887x over previous
import functools

import jax
import jax.numpy as jnp
from jax.experimental import pallas as pl
from jax.experimental.pallas import tpu as pltpu

NG = 128
FX = 128
H = 128
FU = 128
FOUT = 2

_MED = jax.lax.Precision.DEFAULT


def _pick_blk(n):
    for blk in (10000, 4000, 2000, 1000, 500, 250, 200, 125, 100, 50, 25, 20,
                10, 8, 5, 4, 2, 1):
        if n % blk == 0:
            return blk
    return n


def _apply_last(s, cnt, w2, b2):
    s_lo = s - s.astype(jnp.bfloat16).astype(jnp.float32)
    s_ll = s_lo - s_lo.astype(jnp.bfloat16).astype(jnp.float32)
    return (jnp.dot(s, w2, preferred_element_type=jnp.float32)
            + jnp.dot(s_lo, w2, preferred_element_type=jnp.float32)
            + jnp.dot(s_ll, w2, preferred_element_type=jnp.float32)
            + cnt * b2)


def _mlp3(x, w0, b0, w1, b1, w2, b2):
    y = jnp.maximum(
        jnp.dot(x, w0, preferred_element_type=jnp.float32, precision=_MED) + b0,
        0.0)
    y = jnp.maximum(
        jnp.dot(y, w1, preferred_element_type=jnp.float32, precision=_MED) + b1,
        0.0)
    return jnp.dot(y, w2, preferred_element_type=jnp.float32, precision=_MED) + b2


def _body(nb, blk,
          x1_ref, x2_ref, b1_ref, b2_ref,
          e10, e1b0, e11, e1b1, e12, e1b2,
          r10, r1b0, r11, r1b1, r12, r1b2,
          e2a, e2b, e2b0, e21, e2b1, e22, e2b2,
          r20, r2b0, r21, r2b1, r22, r2b2,
          m0, mb0, m1, mb1, m2, mb2,
          o_ref, agg1, u1s, agg2, cnt1, cnt2):
    p = pl.program_id(0)
    i = pl.program_id(1)

    @pl.when(jnp.logical_and(p == 0, i == 0))
    def _():
        agg1[...] = jnp.zeros_like(agg1)
        agg2[...] = jnp.zeros_like(agg2)
        cnt1[...] = jnp.zeros_like(cnt1)
        cnt2[...] = jnp.zeros_like(cnt2)

    @pl.when(p == 0)
    def _():
        x = x1_ref[...]
        y = jnp.maximum(jnp.dot(x, e10[...], preferred_element_type=jnp.float32,
                                precision=_MED) + e1b0[...], 0.0)
        y = jnp.maximum(jnp.dot(y, e11[...], preferred_element_type=jnp.float32,
                                precision=_MED) + e1b1[...], 0.0)
        ids = b1_ref[...].reshape(1, blk)
        oh = (jax.lax.broadcasted_iota(jnp.int32, (NG, blk), 0) == ids
              ).astype(jnp.float32)
        agg1[...] += jnp.dot(oh, y, preferred_element_type=jnp.float32)
        cnt1[...] += jnp.sum(oh, axis=1, keepdims=True)

    @pl.when(jnp.logical_and(p == 1, i == 0))
    def _():
        a1 = _apply_last(agg1[...], cnt1[...], e12[...], e1b2[...])
        u1 = _mlp3(a1, r10[...], r1b0[...], r11[...], r1b1[...],
                   r12[...], r1b2[...])
        u1s[...] = u1.astype(jnp.bfloat16).astype(jnp.float32)

    @pl.when(p == 1)
    def _():
        ids1 = b1_ref[...].reshape(1, blk)
        oh1 = (jax.lax.broadcasted_iota(jnp.int32, (NG, blk), 0) == ids1
               ).astype(jnp.float32)
        u1g = jax.lax.dot_general(oh1, u1s[...], (((0,), (0,)), ((), ())),
                                  preferred_element_type=jnp.float32)
        y = jnp.dot(x2_ref[...], e2a[...], preferred_element_type=jnp.float32,
                    precision=_MED)
        y = y + jnp.dot(u1g, e2b[...], preferred_element_type=jnp.float32,
                        precision=_MED)
        y = jnp.maximum(y + e2b0[...], 0.0)
        y = jnp.maximum(jnp.dot(y, e21[...], preferred_element_type=jnp.float32,
                                precision=_MED) + e2b1[...], 0.0)
        ids2 = b2_ref[...].reshape(1, blk)
        oh2 = (jax.lax.broadcasted_iota(jnp.int32, (NG, blk), 0) == ids2
               ).astype(jnp.float32)
        agg2[...] += jnp.dot(oh2, y, preferred_element_type=jnp.float32)
        cnt2[...] += jnp.sum(oh2, axis=1, keepdims=True)

    @pl.when(jnp.logical_and(p == 1, i == nb - 1))
    def _():
        a2 = _apply_last(agg2[...], cnt2[...], e22[...], e2b2[...])
        u2 = _mlp3(a2, r20[...], r2b0[...], r21[...], r2b1[...],
                   r22[...], r2b2[...])
        o_ref[...] = _mlp3(u2, m0[...], mb0[...], m1[...], mb1[...],
                           m2[...], mb2[...])


def kernel(x1, x2, batch1, batch2,
           ds1_emb_W0, ds1_emb_b0, ds1_emb_W1, ds1_emb_b1, ds1_emb_W2, ds1_emb_b2,
           ds1_red_W0, ds1_red_b0, ds1_red_W1, ds1_red_b1, ds1_red_W2, ds1_red_b2,
           ds2_emb_W0, ds2_emb_b0, ds2_emb_W1, ds2_emb_b1, ds2_emb_W2, ds2_emb_b2,
           ds2_red_W0, ds2_red_b0, ds2_red_W1, ds2_red_b1, ds2_red_W2, ds2_red_b2,
           mlp_W0, mlp_b0, mlp_W1, mlp_b1, mlp_W2, mlp_b2):
    n = x1.shape[0]
    blk = _pick_blk(n)
    nb = n // blk

    b1r = batch1.reshape(nb, 1, blk)
    b2r = batch2.reshape(nb, 1, blk)

    def row(b):
        return b.reshape(1, -1)

    w0a = ds2_emb_W0[:FX]
    w0b = ds2_emb_W0[FX:]

    m2p = jnp.zeros((FU, 128), jnp.float32).at[:, :FOUT].set(mlp_W2)
    mb2p = jnp.zeros((1, 128), jnp.float32).at[:, :FOUT].set(mlp_b2)

    def full(a):
        return pl.BlockSpec(a.shape, lambda p, i: (0,) * a.ndim)

    weights = [ds1_emb_W0, row(ds1_emb_b0), ds1_emb_W1, row(ds1_emb_b1),
               ds1_emb_W2, row(ds1_emb_b2),
               ds1_red_W0, row(ds1_red_b0), ds1_red_W1, row(ds1_red_b1),
               ds1_red_W2, row(ds1_red_b2),
               w0a, w0b, row(ds2_emb_b0), ds2_emb_W1, row(ds2_emb_b1),
               ds2_emb_W2, row(ds2_emb_b2),
               ds2_red_W0, row(ds2_red_b0), ds2_red_W1, row(ds2_red_b1),
               ds2_red_W2, row(ds2_red_b2),
               mlp_W0, row(mlp_b0), mlp_W1, row(mlp_b1), m2p, mb2p]

    in_specs = [
        pl.BlockSpec((blk, FX), lambda p, i: (jnp.where(p == 0, i, 0), 0)),
        pl.BlockSpec((blk, FX), lambda p, i: (jnp.where(p == 1, i, 0), 0)),
        pl.BlockSpec((1, 1, blk), lambda p, i: (i, 0, 0)),
        pl.BlockSpec((1, 1, blk), lambda p, i: (i, 0, 0)),
    ] + [full(w) for w in weights]

    out = pl.pallas_call(
        functools.partial(_body, nb, blk),
        grid=(2, nb),
        in_specs=in_specs,
        out_specs=pl.BlockSpec((NG, 128), lambda p, i: (0, 0)),
        out_shape=jax.ShapeDtypeStruct((NG, 128), jnp.float32),
        scratch_shapes=[
            pltpu.VMEM((NG, H), jnp.float32),
            pltpu.VMEM((NG, FU), jnp.float32),
            pltpu.VMEM((NG, H), jnp.float32),
            pltpu.VMEM((NG, 1), jnp.float32),
            pltpu.VMEM((NG, 1), jnp.float32),
        ],
        compiler_params=pltpu.CompilerParams(
            dimension_semantics=("arbitrary", "arbitrary"),
        ),
    )(x1, x2, b1r, b2r, *weights)
    return out[:, :FOUT]
